# Initial kernel scaffold; baseline (speedup 1.0000x reference)
import jax
import jax.numpy as jnp
from jax.experimental import pallas as pl


def kernel(x, edge_index, W1, b1, W2, b2):
    raise NotImplementedError("write your pallas kernel here")


# speedup vs baseline: 16.7667x; 16.7667x over previous
import functools

import jax
import jax.numpy as jnp
from jax import lax
from jax.experimental import pallas as pl
from jax.experimental.pallas import tpu as pltpu
from jax.experimental.pallas import tpu_sc as plsc

N = 10000
E = 320000
D = 128
U = 1000
NC = 2
NS = 16
NW = NC * NS
EPW = E // NW
K = 80
NCH = EPW // K

_mesh = plsc.VectorSubcoreMesh(core_axis_name="c", subcore_axis_name="s")


def _zero_vmem_2d(ref, rows, cols):
    z = jnp.zeros((16,), jnp.float32)

    @pl.loop(0, rows)
    def _(r):
        for c in range(cols // 16):
            ref[r, pl.ds(c * 16, 16)] = z


@functools.partial(
    pl.kernel,
    out_type=jax.ShapeDtypeStruct((NC, N), jnp.float32),
    mesh=_mesh,
    scratch_types=[
        pltpu.VMEM((NCH, K), jnp.int32),
        pltpu.VMEM((K,), jnp.float32),
        pltpu.VMEM((2000,), jnp.float32),
        pltpu.VMEM_SHARED((N,), jnp.float32),
    ],
)
def _deg_kernel(dst_hbm, out_hbm, idx_v, ones_v, zb_v, acc_sh):
    cid = lax.axis_index("c")
    sid = lax.axis_index("s")
    pltpu.sync_copy(dst_hbm.at[cid, sid], idx_v)

    for i in range(K // 16):
        ones_v[pl.ds(i * 16, 16)] = jnp.ones((16,), jnp.float32)

    @pl.when(sid == 0)
    def _():
        z = jnp.zeros((16,), jnp.float32)

        @pl.loop(0, 125)
        def _(r):
            zb_v[pl.ds(r * 16, 16)] = z

        for i in range(5):
            pltpu.sync_copy(zb_v, acc_sh.at[pl.ds(i * 2000, 2000)])

    plsc.subcore_barrier()

    @pl.loop(0, NCH)
    def _(j):
        pltpu.sync_copy(ones_v, acc_sh.at[idx_v.at[j]], add=True)

    plsc.subcore_barrier()

    @pl.when(sid == 0)
    def _():
        pltpu.sync_copy(acc_sh, out_hbm.at[cid])


@functools.partial(
    pl.kernel,
    out_type=jax.ShapeDtypeStruct((NC, N, D), jnp.float32),
    mesh=_mesh,
    scratch_types=[
        pltpu.VMEM((NCH, K), jnp.int32),
        pltpu.VMEM((NCH, K), jnp.int32),
        pltpu.VMEM((K, D), jnp.float32),
        pltpu.VMEM((16, D), jnp.float32),
        pltpu.VMEM_SHARED((N, D), jnp.float32),
        pltpu.SemaphoreType.DMA,
    ],
)
def _edge_kernel(y_hbm, src_hbm, dst_hbm, out_hbm, src_v, dst_v, buf_v, zb_v,
                 acc_sh, sem):
    cid = lax.axis_index("c")
    sid = lax.axis_index("s")
    pltpu.sync_copy(src_hbm.at[cid, sid], src_v)
    pltpu.sync_copy(dst_hbm.at[cid, sid], dst_v)

    _zero_vmem_2d(zb_v, 16, D)

    @pl.loop(sid, N // 16, step=NS)
    def _(i):
        pltpu.sync_copy(zb_v, acc_sh.at[pl.ds(i * 16, 16)])

    plsc.subcore_barrier()

    @pl.loop(0, NCH)
    def _(j):
        pltpu.async_copy(y_hbm.at[src_v.at[j]], buf_v, sem).wait()
        pltpu.sync_copy(buf_v, acc_sh.at[dst_v.at[j]], add=True)

    plsc.subcore_barrier()

    @pl.loop(sid, N // 16, step=NS)
    def _(i):
        pltpu.sync_copy(acc_sh.at[pl.ds(i * 16, 16)],
                        out_hbm.at[cid, pl.ds(i * 16, 16)])


def _dis_body(d0_ref, d1_ref, dis_ref):
    deg = d0_ref[...] + d1_ref[...] + 1.0
    dis_ref[...] = lax.rsqrt(deg)


def _scale_mm_body(x_ref, w_ref, dis_ref, y_ref):
    xw = jnp.dot(x_ref[...], w_ref[...], preferred_element_type=jnp.float32)
    y_ref[...] = xw * dis_ref[...]


def _mid_body(p0_ref, p1_ref, y_ref, dis_ref, b_ref, w_ref, out_ref):
    dis = dis_ref[...]
    t = dis * (p0_ref[...] + p1_ref[...] + y_ref[...]) + b_ref[...]
    h = jnp.where(t > 0, t, jnp.exp(t) - 1.0)
    hw = jnp.dot(h, w_ref[...], preferred_element_type=jnp.float32)
    out_ref[...] = hw * dis


def _final_body(q0_ref, q1_ref, y_ref, dis_ref, b_ref, out_ref):
    dis = dis_ref[...]
    t = dis * (q0_ref[...] + q1_ref[...] + y_ref[...]) + b_ref[...]
    out_ref[...] = jnp.where(t > 0, t, jnp.exp(t) - 1.0)


def kernel(x, edge_index, W1, b1, W2, b2):
    ei = edge_index.astype(jnp.int32)
    src = ei[0].reshape(NC, NS, NCH, K)
    dst = ei[1].reshape(NC, NS, NCH, K)

    deg_p = _deg_kernel(dst)

    dis = pl.pallas_call(
        _dis_body,
        out_shape=jax.ShapeDtypeStruct((N,), jnp.float32),
    )(deg_p[0], deg_p[1])
    dis2 = dis.reshape(N, 1)

    R = 400
    grid = N // R

    def _rows(i):
        return (i, 0)

    row_spec = pl.BlockSpec((R, D), _rows)
    dis_spec = pl.BlockSpec((R, 1), lambda i: (i, 0))
    full_spec = pl.BlockSpec((D, D), lambda i: (0, 0))
    bias_spec = pl.BlockSpec((1, D), lambda i: (0, 0))

    y1 = pl.pallas_call(
        _scale_mm_body,
        grid=(grid,),
        in_specs=[row_spec, full_spec, dis_spec],
        out_specs=row_spec,
        out_shape=jax.ShapeDtypeStruct((N, D), jnp.float32),
    )(x, W1, dis2)

    p = _edge_kernel(y1, src, dst)

    y2 = pl.pallas_call(
        _mid_body,
        grid=(grid,),
        in_specs=[row_spec, row_spec, row_spec, dis_spec, bias_spec, full_spec],
        out_specs=row_spec,
        out_shape=jax.ShapeDtypeStruct((N, D), jnp.float32),
    )(p[0], p[1], y1, dis2, b1.reshape(1, D), W2)

    q = _edge_kernel(y2, src, dst)

    RT = 128
    top_grid = 1024 // RT
    top_spec = pl.BlockSpec((RT, D), lambda i: (i, 0))
    top_dis = pl.BlockSpec((RT, 1), lambda i: (i, 0))
    top_bias = pl.BlockSpec((1, D), lambda i: (0, 0))
    out_top = pl.pallas_call(
        _final_body,
        grid=(top_grid,),
        in_specs=[top_spec, top_spec, top_spec, top_dis, top_bias],
        out_specs=top_spec,
        out_shape=jax.ShapeDtypeStruct((1024, D), jnp.float32),
    )(q[0, :1024], q[1, :1024], y2[:1024], dis2[:1024], b2.reshape(1, D))

    return jnp.concatenate([out_top[:U], x[U:]], axis=0)
